# baseline (speedup 1.0000x reference)
import functools

import jax
import jax.numpy as jnp
from jax import lax
from jax.experimental import pallas as pl
from jax.experimental.pallas import tpu as pltpu
from jax.experimental.pallas import tpu_sc as plsc

N_NODES = 10000
N_EDGES = 160000
CH = 128
KP = 15
KPAD = 16

NC = 2
NS = 16
NW = NC * NS

EDGE_CHUNK = 64
E_PAD = 163840
EDGES_PER_W = E_PAD // NW
N_CHUNKS = EDGES_PER_W // EDGE_CHUNK

ACC_ROWS = 10240
ROWS_PER_W = ACC_ROWS // NS
PAD_TARGET = N_NODES

C0_CHUNKS = 0
C1_CHUNKS = 160
RING = 2

TC_GRID = 10
H_BN = N_NODES // TC_GRID
E_BR = E_PAD // CH // TC_GRID
E_ROWS = N_EDGES // CH


def _tc_body(f_ref, w_ref, h3_ref, s_ref, t_ref, mu_ref,
             oh_ref, og_ref, ot_ref):
    i = pl.program_id(0)
    f = f_ref[...]
    for k in range(KP):
        wk = w_ref[:, k, :]
        oh_ref[k] = lax.dot_general(
            f, wk, (((1,), (1,)), ((), ())),
            preferred_element_type=jnp.float32)
    hx = h3_ref[0]
    hy = h3_ref[1]
    hz = h3_ref[2]
    best = jnp.full(hx.shape, jnp.inf, jnp.float32)
    bidx = jnp.zeros(hx.shape, jnp.int32)
    for k in range(KP):
        dx = hx - mu_ref[k, 0]
        dy = hy - mu_ref[k, 1]
        dz = hz - mu_ref[k, 2]
        d = jnp.sqrt(dx * dx + dy * dy + dz * dz)
        m = d < best
        best = jnp.where(m, d, best)
        bidx = jnp.where(m, k, bidx)
    row = i * E_BR + lax.broadcasted_iota(jnp.int32, hx.shape, 0)
    valid = row < E_ROWS
    og_ref[...] = jnp.where(valid, bidx * N_NODES + s_ref[...], 0)
    ot_ref[...] = jnp.where(valid, t_ref[...], PAD_TARGET)


def _sc_gather_scatter(h_flat, gidx2d, tgt2d):
    mesh = plsc.VectorSubcoreMesh(core_axis_name="c", subcore_axis_name="s")
    cmax = max(C0_CHUNKS, C1_CHUNKS)

    @functools.partial(
        pl.kernel,
        out_type=jax.ShapeDtypeStruct((NC * ACC_ROWS, CH), jnp.float32),
        mesh=mesh,
        scratch_types=[
            pltpu.VMEM((cmax, EDGE_CHUNK), jnp.int32),
            pltpu.VMEM((RING, EDGE_CHUNK), jnp.int32),
            pltpu.VMEM((RING, EDGE_CHUNK, CH), jnp.float32),
            pltpu.VMEM_SHARED((ACC_ROWS, CH), jnp.float32),
        ] + [pltpu.SemaphoreType.DMA] * (2 * RING),
    )
    def sc_kernel(h_hbm, gidx_hbm, tgt_hbm, out_hbm, idx_all, tgt_all,
                  rows, acc, *sems):
        cid = lax.axis_index("c")
        sid = lax.axis_index("s")
        gsem = list(sems[:RING])
        ssem = list(sems[RING:])

        @pl.loop(0, EDGE_CHUNK)
        def _(i):
            for j in range(CH // 16):
                rows[0, i, pl.ds(j * 16, 16)] = jnp.zeros((16,), jnp.float32)

        @pl.loop(0, ROWS_PER_W // EDGE_CHUNK)
        def _(t):
            pltpu.sync_copy(
                rows.at[0],
                acc.at[pl.ds(sid * ROWS_PER_W + t * EDGE_CHUNK, EDGE_CHUNK)])

        def pipeline(base_row, count):
            pltpu.sync_copy(gidx_hbm.at[pl.ds(base_row, count)],
                            idx_all.at[pl.ds(0, count)])
            gh = [None] * count
            th = [None] * count
            sh = [None] * count

            def gissue(c):
                b = c % RING
                th[c] = pltpu.async_copy(tgt_hbm.at[pl.ds(base_row + c, 1)],
                                         tgt_all.at[pl.ds(b, 1)], gsem[b])
                gh[c] = pltpu.async_copy(h_hbm.at[idx_all.at[c]],
                                         rows.at[b], gsem[b])

            for c in range(min(RING - 1, count)):
                gissue(c)
            for c in range(count):
                gh[c].wait()
                th[c].wait()
                sh[c] = pltpu.async_copy(rows.at[c % RING],
                                         acc.at[tgt_all.at[c % RING]],
                                         ssem[c % RING], add=True)
                if c + RING - 1 < count:
                    if c - 1 >= 0:
                        sh[c - 1].wait()
                    gissue(c + RING - 1)
            for c in range(max(0, count - RING), count):
                sh[c].wait()

        if C0_CHUNKS > 0:
            @pl.when(cid == 0)
            def _():
                pipeline(sid * C0_CHUNKS, C0_CHUNKS)
        if C1_CHUNKS > 0:
            @pl.when(cid == 1)
            def _():
                pipeline(NS * C0_CHUNKS + sid * C1_CHUNKS, C1_CHUNKS)

        plsc.subcore_barrier()
        pltpu.sync_copy(
            acc.at[pl.ds(sid * ROWS_PER_W, ROWS_PER_W)],
            out_hbm.at[pl.ds(cid * ACC_ROWS + sid * ROWS_PER_W, ROWS_PER_W)])

    return sc_kernel(h_flat, gidx2d, tgt2d)


def kernel(source, target, features, hood_coords, w, mu):
    n = features.shape[0]

    h3 = hood_coords.T.reshape(3, E_ROWS, CH)
    src2 = source.reshape(E_ROWS, CH)
    tgt2 = target.reshape(E_ROWS, CH)
    h, gidx2, tgtp2 = pl.pallas_call(
        _tc_body,
        grid=(TC_GRID,),
        in_specs=[
            pl.BlockSpec((H_BN, CH), lambda i: (i, 0)),
            pl.BlockSpec((CH, KP, CH), lambda i: (0, 0, 0)),
            pl.BlockSpec((3, E_BR, CH), lambda i: (0, i, 0)),
            pl.BlockSpec((E_BR, CH), lambda i: (i, 0)),
            pl.BlockSpec((E_BR, CH), lambda i: (i, 0)),
            pl.BlockSpec(memory_space=pltpu.SMEM),
        ],
        out_specs=[
            pl.BlockSpec((KP, H_BN, CH), lambda i: (0, i, 0)),
            pl.BlockSpec((E_BR, CH), lambda i: (i, 0)),
            pl.BlockSpec((E_BR, CH), lambda i: (i, 0)),
        ],
        out_shape=[
            jax.ShapeDtypeStruct((KP, N_NODES, CH), jnp.float32),
            jax.ShapeDtypeStruct((E_PAD // CH, CH), jnp.int32),
            jax.ShapeDtypeStruct((E_PAD // CH, CH), jnp.int32),
        ],
    )(features, w, h3, src2, tgt2, mu[0])
    h_flat = h.reshape(KP * N_NODES, CH)
    tgt2d = tgtp2.reshape(E_PAD // EDGE_CHUNK, EDGE_CHUNK)
    gidx2d = gidx2.reshape(E_PAD // EDGE_CHUNK, EDGE_CHUNK)

    partials = _sc_gather_scatter(h_flat, gidx2d, tgt2d)

    return partials[:n] + partials[ACC_ROWS:ACC_ROWS + n]


# speedup vs baseline: 1.2055x; 1.2055x over previous
import functools

import jax
import jax.numpy as jnp
from jax import lax
from jax.experimental import pallas as pl
from jax.experimental.pallas import tpu as pltpu
from jax.experimental.pallas import tpu_sc as plsc

N_NODES = 10000
N_EDGES = 160000
CH = 128
KP = 15
KPAD = 16

NC = 2
NS = 16
NW = NC * NS

EDGE_CHUNK = 64
E_PAD = 163840
EDGES_PER_W = E_PAD // NW
N_CHUNKS = EDGES_PER_W // EDGE_CHUNK

ACC_ROWS = 10240
ROWS_PER_W = ACC_ROWS // NS
PAD_TARGET = N_NODES

C0_CHUNKS = 80
C1_CHUNKS = 80
RING = 2

TC_GRID = 10
H_BN = N_NODES // TC_GRID
E_BR = E_PAD // CH // TC_GRID
E_ROWS = N_EDGES // CH


def _tc_body(f_ref, w_ref, h3_ref, s_ref, t_ref, mu_ref,
             oh_ref, og_ref, ot_ref):
    i = pl.program_id(0)
    f = f_ref[...]
    for k in range(KP):
        wk = w_ref[:, k, :]
        oh_ref[k] = lax.dot_general(
            f, wk, (((1,), (1,)), ((), ())),
            preferred_element_type=jnp.float32)
    hx = h3_ref[0]
    hy = h3_ref[1]
    hz = h3_ref[2]
    best = jnp.full(hx.shape, jnp.inf, jnp.float32)
    bidx = jnp.zeros(hx.shape, jnp.int32)
    for k in range(KP):
        dx = hx - mu_ref[k, 0]
        dy = hy - mu_ref[k, 1]
        dz = hz - mu_ref[k, 2]
        d = jnp.sqrt(dx * dx + dy * dy + dz * dz)
        m = d < best
        best = jnp.where(m, d, best)
        bidx = jnp.where(m, k, bidx)
    row = i * E_BR + lax.broadcasted_iota(jnp.int32, hx.shape, 0)
    valid = row < E_ROWS
    og_ref[...] = jnp.where(valid, bidx * N_NODES + s_ref[...], 0)
    ot_ref[...] = jnp.where(valid, t_ref[...], PAD_TARGET)


def _sc_gather_scatter(h_flat, gidx2d, tgt2d):
    mesh = plsc.VectorSubcoreMesh(core_axis_name="c", subcore_axis_name="s")
    cmax = max(C0_CHUNKS, C1_CHUNKS)

    @functools.partial(
        pl.kernel,
        out_type=jax.ShapeDtypeStruct((NC * ACC_ROWS, CH), jnp.float32),
        mesh=mesh,
        scratch_types=[
            pltpu.VMEM((cmax, EDGE_CHUNK), jnp.int32),
            pltpu.VMEM((RING, EDGE_CHUNK), jnp.int32),
            pltpu.VMEM((RING, EDGE_CHUNK, CH), jnp.float32),
            pltpu.VMEM_SHARED((ACC_ROWS, CH), jnp.float32),
        ] + [pltpu.SemaphoreType.DMA] * (2 * RING),
    )
    def sc_kernel(h_hbm, gidx_hbm, tgt_hbm, out_hbm, idx_all, tgt_all,
                  rows, acc, *sems):
        cid = lax.axis_index("c")
        sid = lax.axis_index("s")
        gsem = list(sems[:RING])
        ssem = list(sems[RING:])

        @pl.loop(0, EDGE_CHUNK)
        def _(i):
            for j in range(CH // 16):
                rows[0, i, pl.ds(j * 16, 16)] = jnp.zeros((16,), jnp.float32)

        @pl.loop(0, ROWS_PER_W // EDGE_CHUNK)
        def _(t):
            pltpu.sync_copy(
                rows.at[0],
                acc.at[pl.ds(sid * ROWS_PER_W + t * EDGE_CHUNK, EDGE_CHUNK)])

        def pipeline(base_row, count):
            pltpu.sync_copy(gidx_hbm.at[pl.ds(base_row, count)],
                            idx_all.at[pl.ds(0, count)])
            gh = [None] * count
            th = [None] * count
            sh = [None] * count

            def gissue(c):
                b = c % RING
                th[c] = pltpu.async_copy(tgt_hbm.at[pl.ds(base_row + c, 1)],
                                         tgt_all.at[pl.ds(b, 1)], gsem[b])
                gh[c] = pltpu.async_copy(h_hbm.at[idx_all.at[c]],
                                         rows.at[b], gsem[b])

            for c in range(min(RING - 1, count)):
                gissue(c)
            for c in range(count):
                gh[c].wait()
                th[c].wait()
                sh[c] = pltpu.async_copy(rows.at[c % RING],
                                         acc.at[tgt_all.at[c % RING]],
                                         ssem[c % RING], add=True)
                if c + RING - 1 < count:
                    if c - 1 >= 0:
                        sh[c - 1].wait()
                    gissue(c + RING - 1)
            for c in range(max(0, count - RING), count):
                sh[c].wait()

        if C0_CHUNKS > 0:
            @pl.when(cid == 0)
            def _():
                pipeline(sid * C0_CHUNKS, C0_CHUNKS)
        if C1_CHUNKS > 0:
            @pl.when(cid == 1)
            def _():
                pipeline(NS * C0_CHUNKS + sid * C1_CHUNKS, C1_CHUNKS)

        plsc.subcore_barrier()
        pltpu.sync_copy(
            acc.at[pl.ds(sid * ROWS_PER_W, ROWS_PER_W)],
            out_hbm.at[pl.ds(cid * ACC_ROWS + sid * ROWS_PER_W, ROWS_PER_W)])

    return sc_kernel(h_flat, gidx2d, tgt2d)


def kernel(source, target, features, hood_coords, w, mu):
    n = features.shape[0]

    h3 = hood_coords.T.reshape(3, E_ROWS, CH)
    src2 = source.reshape(E_ROWS, CH)
    tgt2 = target.reshape(E_ROWS, CH)
    h, gidx2, tgtp2 = pl.pallas_call(
        _tc_body,
        grid=(TC_GRID,),
        in_specs=[
            pl.BlockSpec((H_BN, CH), lambda i: (i, 0)),
            pl.BlockSpec((CH, KP, CH), lambda i: (0, 0, 0)),
            pl.BlockSpec((3, E_BR, CH), lambda i: (0, i, 0)),
            pl.BlockSpec((E_BR, CH), lambda i: (i, 0)),
            pl.BlockSpec((E_BR, CH), lambda i: (i, 0)),
            pl.BlockSpec(memory_space=pltpu.SMEM),
        ],
        out_specs=[
            pl.BlockSpec((KP, H_BN, CH), lambda i: (0, i, 0)),
            pl.BlockSpec((E_BR, CH), lambda i: (i, 0)),
            pl.BlockSpec((E_BR, CH), lambda i: (i, 0)),
        ],
        out_shape=[
            jax.ShapeDtypeStruct((KP, N_NODES, CH), jnp.float32),
            jax.ShapeDtypeStruct((E_PAD // CH, CH), jnp.int32),
            jax.ShapeDtypeStruct((E_PAD // CH, CH), jnp.int32),
        ],
    )(features, w, h3, src2, tgt2, mu[0])
    h_flat = h.reshape(KP * N_NODES, CH)
    tgt2d = tgtp2.reshape(E_PAD // EDGE_CHUNK, EDGE_CHUNK)
    gidx2d = gidx2.reshape(E_PAD // EDGE_CHUNK, EDGE_CHUNK)

    partials = _sc_gather_scatter(h_flat, gidx2d, tgt2d)

    return partials[:n] + partials[ACC_ROWS:ACC_ROWS + n]


# speedup vs baseline: 1.2877x; 1.0681x over previous
import functools

import jax
import jax.numpy as jnp
from jax import lax
from jax.experimental import pallas as pl
from jax.experimental.pallas import tpu as pltpu
from jax.experimental.pallas import tpu_sc as plsc

N_NODES = 10000
N_EDGES = 160000
CH = 128
KP = 15
KPAD = 16

NC = 2
NS = 16
NW = NC * NS

EDGE_CHUNK = 64
E_PAD = 163840
EDGES_PER_W = E_PAD // NW
N_CHUNKS = EDGES_PER_W // EDGE_CHUNK

ACC_ROWS = 10240
ROWS_PER_W = ACC_ROWS // NS
PAD_TARGET = N_NODES

C0_CHUNKS = 80
C1_CHUNKS = 80
RING = 3

TC_GRID = 10
H_BN = N_NODES // TC_GRID
E_BR = E_PAD // CH // TC_GRID
E_ROWS = N_EDGES // CH


def _tc_body(f_ref, w_ref, h3_ref, s_ref, t_ref, mu_ref,
             oh_ref, og_ref, ot_ref):
    i = pl.program_id(0)
    f = f_ref[...]
    for k in range(KP):
        wk = w_ref[:, k, :]
        oh_ref[k] = lax.dot_general(
            f, wk, (((1,), (1,)), ((), ())),
            preferred_element_type=jnp.float32)
    hx = h3_ref[0]
    hy = h3_ref[1]
    hz = h3_ref[2]
    best = jnp.full(hx.shape, jnp.inf, jnp.float32)
    bidx = jnp.zeros(hx.shape, jnp.int32)
    for k in range(KP):
        dx = hx - mu_ref[k, 0]
        dy = hy - mu_ref[k, 1]
        dz = hz - mu_ref[k, 2]
        d = jnp.sqrt(dx * dx + dy * dy + dz * dz)
        m = d < best
        best = jnp.where(m, d, best)
        bidx = jnp.where(m, k, bidx)
    row = i * E_BR + lax.broadcasted_iota(jnp.int32, hx.shape, 0)
    valid = row < E_ROWS
    og_ref[...] = jnp.where(valid, bidx * N_NODES + s_ref[...], 0)
    ot_ref[...] = jnp.where(valid, t_ref[...], PAD_TARGET)


def _sc_gather_scatter(h_flat, gidx2d, tgt2d):
    mesh = plsc.VectorSubcoreMesh(core_axis_name="c", subcore_axis_name="s")
    cmax = max(C0_CHUNKS, C1_CHUNKS)

    @functools.partial(
        pl.kernel,
        out_type=jax.ShapeDtypeStruct((NC * ACC_ROWS, CH), jnp.float32),
        mesh=mesh,
        scratch_types=[
            pltpu.VMEM((cmax, EDGE_CHUNK), jnp.int32),
            pltpu.VMEM((RING, EDGE_CHUNK), jnp.int32),
            pltpu.VMEM((RING, EDGE_CHUNK, CH), jnp.float32),
            pltpu.VMEM_SHARED((ACC_ROWS, CH), jnp.float32),
        ] + [pltpu.SemaphoreType.DMA] * (2 * RING),
    )
    def sc_kernel(h_hbm, gidx_hbm, tgt_hbm, out_hbm, idx_all, tgt_all,
                  rows, acc, *sems):
        cid = lax.axis_index("c")
        sid = lax.axis_index("s")
        gsem = list(sems[:RING])
        ssem = list(sems[RING:])

        @pl.loop(0, EDGE_CHUNK)
        def _(i):
            for j in range(CH // 16):
                rows[0, i, pl.ds(j * 16, 16)] = jnp.zeros((16,), jnp.float32)

        @pl.loop(0, ROWS_PER_W // EDGE_CHUNK)
        def _(t):
            pltpu.sync_copy(
                rows.at[0],
                acc.at[pl.ds(sid * ROWS_PER_W + t * EDGE_CHUNK, EDGE_CHUNK)])

        def pipeline(base_row, count):
            pltpu.sync_copy(gidx_hbm.at[pl.ds(base_row, count)],
                            idx_all.at[pl.ds(0, count)])
            gh = [None] * count
            th = [None] * count
            sh = [None] * count

            def gissue(c):
                b = c % RING
                th[c] = pltpu.async_copy(tgt_hbm.at[pl.ds(base_row + c, 1)],
                                         tgt_all.at[pl.ds(b, 1)], gsem[b])
                gh[c] = pltpu.async_copy(h_hbm.at[idx_all.at[c]],
                                         rows.at[b], gsem[b])

            for c in range(min(RING - 1, count)):
                gissue(c)
            for c in range(count):
                gh[c].wait()
                th[c].wait()
                sh[c] = pltpu.async_copy(rows.at[c % RING],
                                         acc.at[tgt_all.at[c % RING]],
                                         ssem[c % RING], add=True)
                if c + RING - 1 < count:
                    if c - 1 >= 0:
                        sh[c - 1].wait()
                    gissue(c + RING - 1)
            for c in range(max(0, count - RING), count):
                sh[c].wait()

        if C0_CHUNKS > 0:
            @pl.when(cid == 0)
            def _():
                pipeline(sid * C0_CHUNKS, C0_CHUNKS)
        if C1_CHUNKS > 0:
            @pl.when(cid == 1)
            def _():
                pipeline(NS * C0_CHUNKS + sid * C1_CHUNKS, C1_CHUNKS)

        plsc.subcore_barrier()
        pltpu.sync_copy(
            acc.at[pl.ds(sid * ROWS_PER_W, ROWS_PER_W)],
            out_hbm.at[pl.ds(cid * ACC_ROWS + sid * ROWS_PER_W, ROWS_PER_W)])

    return sc_kernel(h_flat, gidx2d, tgt2d)


def kernel(source, target, features, hood_coords, w, mu):
    n = features.shape[0]

    h3 = hood_coords.T.reshape(3, E_ROWS, CH)
    src2 = source.reshape(E_ROWS, CH)
    tgt2 = target.reshape(E_ROWS, CH)
    h, gidx2, tgtp2 = pl.pallas_call(
        _tc_body,
        grid=(TC_GRID,),
        in_specs=[
            pl.BlockSpec((H_BN, CH), lambda i: (i, 0)),
            pl.BlockSpec((CH, KP, CH), lambda i: (0, 0, 0)),
            pl.BlockSpec((3, E_BR, CH), lambda i: (0, i, 0)),
            pl.BlockSpec((E_BR, CH), lambda i: (i, 0)),
            pl.BlockSpec((E_BR, CH), lambda i: (i, 0)),
            pl.BlockSpec(memory_space=pltpu.SMEM),
        ],
        out_specs=[
            pl.BlockSpec((KP, H_BN, CH), lambda i: (0, i, 0)),
            pl.BlockSpec((E_BR, CH), lambda i: (i, 0)),
            pl.BlockSpec((E_BR, CH), lambda i: (i, 0)),
        ],
        out_shape=[
            jax.ShapeDtypeStruct((KP, N_NODES, CH), jnp.float32),
            jax.ShapeDtypeStruct((E_PAD // CH, CH), jnp.int32),
            jax.ShapeDtypeStruct((E_PAD // CH, CH), jnp.int32),
        ],
    )(features, w, h3, src2, tgt2, mu[0])
    h_flat = h.reshape(KP * N_NODES, CH)
    tgt2d = tgtp2.reshape(E_PAD // EDGE_CHUNK, EDGE_CHUNK)
    gidx2d = gidx2.reshape(E_PAD // EDGE_CHUNK, EDGE_CHUNK)

    partials = _sc_gather_scatter(h_flat, gidx2d, tgt2d)

    return partials[:n] + partials[ACC_ROWS:ACC_ROWS + n]


# speedup vs baseline: 1.2911x; 1.0026x over previous
import functools

import jax
import jax.numpy as jnp
from jax import lax
from jax.experimental import pallas as pl
from jax.experimental.pallas import tpu as pltpu
from jax.experimental.pallas import tpu_sc as plsc

N_NODES = 10000
N_EDGES = 160000
CH = 128
KP = 15
KPAD = 16

NC = 2
NS = 16
NW = NC * NS

EDGE_CHUNK = 64
E_PAD = 163840
EDGES_PER_W = E_PAD // NW
N_CHUNKS = EDGES_PER_W // EDGE_CHUNK

ACC_ROWS = 10240
ROWS_PER_W = ACC_ROWS // NS
PAD_TARGET = N_NODES

C0_CHUNKS = 80
C1_CHUNKS = 80
RING = 3

TC_GRID = 10
H_BN = N_NODES // TC_GRID
E_BR = E_PAD // CH // TC_GRID
E_ROWS = N_EDGES // CH


def _tc_body(f_ref, w_ref, h3_ref, s_ref, t_ref, mu_ref,
             oh_ref, og_ref, ot_ref):
    i = pl.program_id(0)
    f = f_ref[...]
    for k in range(KP):
        wk = w_ref[:, k, :]
        oh_ref[k] = lax.dot_general(
            f, wk, (((1,), (1,)), ((), ())),
            preferred_element_type=jnp.float32)
    hx = h3_ref[0]
    hy = h3_ref[1]
    hz = h3_ref[2]
    best = jnp.full(hx.shape, jnp.inf, jnp.float32)
    bidx = jnp.zeros(hx.shape, jnp.int32)
    for k in range(KP):
        dx = hx - mu_ref[k, 0]
        dy = hy - mu_ref[k, 1]
        dz = hz - mu_ref[k, 2]
        d = jnp.sqrt(dx * dx + dy * dy + dz * dz)
        m = d < best
        best = jnp.where(m, d, best)
        bidx = jnp.where(m, k, bidx)
    row = i * E_BR + lax.broadcasted_iota(jnp.int32, hx.shape, 0)
    valid = row < E_ROWS
    og_ref[...] = jnp.where(valid, bidx * N_NODES + s_ref[...], 0)
    ot_ref[...] = jnp.where(valid, t_ref[...], PAD_TARGET)


def _sc_gather_scatter(h_flat, gidx2d, tgt2d):
    mesh = plsc.VectorSubcoreMesh(core_axis_name="c", subcore_axis_name="s")
    cmax = max(C0_CHUNKS, C1_CHUNKS)

    @functools.partial(
        pl.kernel,
        out_type=jax.ShapeDtypeStruct((NC * ACC_ROWS, CH), jnp.float32),
        mesh=mesh,
        scratch_types=[
            pltpu.VMEM((cmax, EDGE_CHUNK), jnp.int32),
            pltpu.VMEM((RING, EDGE_CHUNK), jnp.int32),
            pltpu.VMEM((RING, EDGE_CHUNK, CH), jnp.float32),
            pltpu.VMEM_SHARED((ACC_ROWS, CH), jnp.float32),
        ] + [pltpu.SemaphoreType.DMA] * (2 * RING),
    )
    def sc_kernel(h_hbm, gidx_hbm, tgt_hbm, out_hbm, idx_all, tgt_all,
                  rows, acc, *sems):
        cid = lax.axis_index("c")
        sid = lax.axis_index("s")
        gsem = list(sems[:RING])
        ssem = list(sems[RING:])

        @pl.loop(0, EDGE_CHUNK)
        def _(i):
            for j in range(CH // 16):
                rows[0, i, pl.ds(j * 16, 16)] = jnp.zeros((16,), jnp.float32)

        @pl.loop(0, ROWS_PER_W // EDGE_CHUNK)
        def _(t):
            pltpu.sync_copy(
                rows.at[0],
                acc.at[pl.ds(sid * ROWS_PER_W + t * EDGE_CHUNK, EDGE_CHUNK)])

        def pipeline(base_row, count):
            pltpu.sync_copy(gidx_hbm.at[pl.ds(base_row, count)],
                            idx_all.at[pl.ds(0, count)])
            gh = [None] * count
            th = [None] * count
            sh = [None] * count

            def gissue(c):
                b = c % RING
                th[c] = pltpu.async_copy(tgt_hbm.at[pl.ds(base_row + c, 1)],
                                         tgt_all.at[pl.ds(b, 1)], gsem[b])
                gh[c] = pltpu.async_copy(h_hbm.at[idx_all.at[c]],
                                         rows.at[b], gsem[b])

            for c in range(min(RING - 1, count)):
                gissue(c)
            for c in range(count):
                gh[c].wait()
                th[c].wait()
                sh[c] = pltpu.async_copy(rows.at[c % RING],
                                         acc.at[tgt_all.at[c % RING]],
                                         ssem[c % RING], add=True)
                if c + RING - 1 < count:
                    if c - 1 >= 0:
                        sh[c - 1].wait()
                    gissue(c + RING - 1)
            for c in range(max(0, count - RING), count):
                sh[c].wait()

        pipeline((cid * NS + sid) * C0_CHUNKS, C0_CHUNKS)

        plsc.subcore_barrier()
        pltpu.sync_copy(
            acc.at[pl.ds(sid * ROWS_PER_W, ROWS_PER_W)],
            out_hbm.at[pl.ds(cid * ACC_ROWS + sid * ROWS_PER_W, ROWS_PER_W)])

    return sc_kernel(h_flat, gidx2d, tgt2d)


def kernel(source, target, features, hood_coords, w, mu):
    n = features.shape[0]

    h3 = hood_coords.T.reshape(3, E_ROWS, CH)
    src2 = source.reshape(E_ROWS, CH)
    tgt2 = target.reshape(E_ROWS, CH)
    h, gidx2, tgtp2 = pl.pallas_call(
        _tc_body,
        grid=(TC_GRID,),
        in_specs=[
            pl.BlockSpec((H_BN, CH), lambda i: (i, 0)),
            pl.BlockSpec((CH, KP, CH), lambda i: (0, 0, 0)),
            pl.BlockSpec((3, E_BR, CH), lambda i: (0, i, 0)),
            pl.BlockSpec((E_BR, CH), lambda i: (i, 0)),
            pl.BlockSpec((E_BR, CH), lambda i: (i, 0)),
            pl.BlockSpec(memory_space=pltpu.SMEM),
        ],
        out_specs=[
            pl.BlockSpec((KP, H_BN, CH), lambda i: (0, i, 0)),
            pl.BlockSpec((E_BR, CH), lambda i: (i, 0)),
            pl.BlockSpec((E_BR, CH), lambda i: (i, 0)),
        ],
        out_shape=[
            jax.ShapeDtypeStruct((KP, N_NODES, CH), jnp.float32),
            jax.ShapeDtypeStruct((E_PAD // CH, CH), jnp.int32),
            jax.ShapeDtypeStruct((E_PAD // CH, CH), jnp.int32),
        ],
    )(features, w, h3, src2, tgt2, mu[0])
    h_flat = h.reshape(KP * N_NODES, CH)
    tgt2d = tgtp2.reshape(E_PAD // EDGE_CHUNK, EDGE_CHUNK)
    gidx2d = gidx2.reshape(E_PAD // EDGE_CHUNK, EDGE_CHUNK)

    partials = _sc_gather_scatter(h_flat, gidx2d, tgt2d)

    return partials[:n] + partials[ACC_ROWS:ACC_ROWS + n]
